# Initial kernel scaffold; baseline (speedup 1.0000x reference)
import jax
import jax.numpy as jnp
from jax.experimental import pallas as pl


def kernel(mask_scores, pred_mask_boxes_cat, gt_association_0, gt_association_1, gt_labels_0, gt_labels_1, gt_masks_0, gt_masks_1):
    raise NotImplementedError("write your pallas kernel here")


# speedup vs baseline: 13.4890x; 13.4890x over previous
import functools

import jax
import jax.numpy as jnp
from jax import lax
from jax.experimental import pallas as pl
from jax.experimental.pallas import tpu as pltpu
from jax.experimental.pallas import tpu_sc as plsc

CROP = 28
LANES = 16


def _splat(x, dtype=jnp.int32):
    return jnp.broadcast_to(jnp.asarray(x, dtype), (LANES,))


def _make_sc_call(n_total, n_per, c, g, h):
    info = plsc.get_sparse_core_info()
    nc, ns = info.num_cores, info.num_subcores
    nw = nc * ns
    chunk = 8
    nchunks = n_per // chunk
    area = CROP * CROP

    mesh = plsc.VectorSubcoreMesh(core_axis_name="c", subcore_axis_name="s")

    @functools.partial(
        pl.kernel,
        out_type=(
            jax.ShapeDtypeStruct((n_total * area,), jnp.float32),
            jax.ShapeDtypeStruct((n_total * area,), jnp.float32),
            jax.ShapeDtypeStruct((n_total,), jnp.int32),
        ),
        mesh=mesh,
        compiler_params=pltpu.CompilerParams(needs_layout_passes=False),
        scratch_types=[
            pltpu.VMEM((LANES,), jnp.int32),
            pltpu.VMEM((4 * chunk,), jnp.float32),
            pltpu.VMEM((h,), jnp.int32),
            pltpu.VMEM((112,), jnp.int32),
            pltpu.VMEM((112,), jnp.int32),
            pltpu.VMEM((112, h), jnp.float32),
            pltpu.VMEM((112, h), jnp.float32),
            pltpu.VMEM((chunk * area,), jnp.float32),
            pltpu.VMEM((chunk * area,), jnp.float32),
            pltpu.VMEM((chunk,), jnp.int32),
            pltpu.SemaphoreType.DMA,
            pltpu.SemaphoreType.DMA,
            pltpu.SemaphoreType.DMA,
        ],
    )
    def sc_call(pred_tbl, boxes, assoc0, assoc1, lab0, lab1, gt0, gt1,
                out_pred, out_crop, out_lab,
                assoc_v, boxes_v, lab_tbl, ridxA, ridxB, rowsA, rowsB,
                pred_buf, stage, lab_stage,
                semA, semB, semP):
        wid = lax.axis_index("s") * nc + lax.axis_index("c")
        iota = jnp.arange(LANES, dtype=jnp.int32)
        m8 = iota < chunk
        idx8 = jnp.minimum(iota, chunk - 1)

        for s in range(2):
            assoc_hbm = assoc0 if s == 0 else assoc1
            lab_hbm = lab0 if s == 0 else lab1
            gt_tbl = gt0 if s == 0 else gt1
            pltpu.sync_copy(lab_hbm, lab_tbl)
            my_chunks = (nchunks + nw - 1 - wid) // nw

            def chunk_body(n, carry):
                cid = wid + nw * n
                base = cid * chunk
                gbase = s * n_per + base
                pltpu.sync_copy(assoc_hbm.at[pl.ds(base, chunk)],
                                assoc_v.at[pl.ds(0, chunk)])
                pltpu.sync_copy(boxes.at[pl.ds(gbase * 4, 4 * chunk)], boxes_v)
                av16 = assoc_v[...]
                bxi0 = boxes_v[pl.ds(0, LANES)].astype(jnp.int32)
                bxi1 = boxes_v[pl.ds(LANES, LANES)].astype(jnp.int32)

                a16 = jnp.clip(av16, 0, g - 1)
                l16 = plsc.load_gather(lab_tbl, [a16])
                l16 = jnp.clip(l16, 0, c - 1)
                plsc.store_scatter(lab_stage, [idx8], l16, mask=m8)

                pidx16 = ((_splat(gbase) + iota) * c + l16) * area
                preds = []
                for b in range(chunk):
                    off = pl.multiple_of(pidx16[b], 8)
                    preds.append(pltpu.async_copy(
                        pred_tbl.at[pl.ds(off, area)],
                        pred_buf.at[pl.ds(b * area, area)], semP))

                for b in range(chunk):
                    tgt = ridxA if b < 4 else ridxB
                    off = (b % 4) * CROP
                    g_s = jnp.clip(av16[b], 0, g - 1)
                    bxh = bxi0 if b < 4 else bxi1
                    y_s = jnp.clip(bxh[(4 * b + 1) % LANES], 0, h - CROP)
                    r0 = _splat(g_s * h + y_s) + iota
                    plsc.store_scatter(tgt, [_splat(off) + iota], r0)
                    plsc.store_scatter(tgt, [_splat(off + 12) + iota], r0 + 12)

                cA = pltpu.async_copy(gt_tbl.at[ridxA], rowsA, semA)
                cB = pltpu.async_copy(gt_tbl.at[ridxB], rowsB, semB)
                pltpu.sync_copy(lab_stage, out_lab.at[pl.ds(gbase, chunk)])
                for cP in preds:
                    cP.wait()
                pltpu.sync_copy(
                    pred_buf,
                    out_pred.at[pl.ds(pl.multiple_of(gbase * area, 8),
                                      chunk * area)])
                cA.wait()
                cB.wait()

                for b in range(chunk):
                    rows = rowsA if b < 4 else rowsB
                    bxh = bxi0 if b < 4 else bxi1
                    x_s = jnp.clip(bxh[(4 * b) % LANES], 0, h - CROP)
                    x16 = _splat(x_s)

                    def row_body(r, x16c):
                        rowid = _splat((b % 4) * CROP + r)
                        lo = plsc.load_gather(rows, [rowid, x16c + iota])
                        plsc.store_scatter(
                            stage, [_splat(b * area + r * CROP) + iota], lo)
                        hi = plsc.load_gather(rows,
                                              [rowid, x16c + 12 + iota])
                        plsc.store_scatter(
                            stage, [_splat(b * area + r * CROP + 12) + iota],
                            hi)
                        return x16c

                    lax.fori_loop(0, CROP, row_body, x16)

                pltpu.sync_copy(
                    stage,
                    out_crop.at[pl.ds(pl.multiple_of(gbase * area, 8),
                                      chunk * area)])
                return carry

            lax.fori_loop(0, my_chunks, chunk_body, 0)

    return sc_call


def kernel(mask_scores, pred_mask_boxes_cat, gt_association_0, gt_association_1,
           gt_labels_0, gt_labels_1, gt_masks_0, gt_masks_1):
    n_total, c = mask_scores.shape[:2]
    n_per = gt_association_0.shape[0]
    g, h = gt_masks_0.shape[:2]
    area = CROP * CROP

    pred_tbl = mask_scores.reshape(-1)
    boxes = pred_mask_boxes_cat.reshape(-1)
    lab0 = jnp.pad(gt_labels_0, (0, h - g))
    lab1 = jnp.pad(gt_labels_1, (0, h - g))
    gt0 = gt_masks_0.reshape(g * h, h)
    gt1 = gt_masks_1.reshape(g * h, h)

    sc_call = _make_sc_call(n_total, n_per, c, g, h)
    out_pred, out_crop, out_lab = sc_call(
        pred_tbl, boxes, gt_association_0, gt_association_1,
        lab0, lab1, gt0, gt1)

    out_pred = out_pred.reshape(n_total, area)
    out_crop = out_crop.reshape(n_total, area)
    pred0 = out_pred[:n_per].reshape(n_per, CROP, CROP)
    pred1 = out_pred[n_per:].reshape(n_per, CROP, CROP)
    crop0 = out_crop[:n_per].reshape(n_per, CROP, CROP)
    crop1 = out_crop[n_per:].reshape(n_per, CROP, CROP)
    return ((pred0, pred1), (crop0, crop1),
            (out_lab[:n_per], out_lab[n_per:]))
